# baseline (speedup 1.0000x reference)
import functools

import jax
import jax.numpy as jnp
from jax import lax
from jax.experimental import pallas as pl
from jax.experimental.pallas import tpu as pltpu
from jax.experimental.pallas import tpu_sc as plsc

N, L, H = 4096, 50, 128
NC, NS, LANES = 2, 16, 16
NW = NC * NS
RPW = N // NW

BLOCK_N = 512

_MESH = plsc.VectorSubcoreMesh(core_axis_name="c", subcore_axis_name="s")


def _flat_indices(vehidx_hbm, idx_v, base):
    pltpu.sync_copy(vehidx_hbm.at[pl.ds(base, RPW)], idx_v)
    for g in range(RPW // LANES):
        sl = pl.ds(g * LANES, LANES)
        row = base + g * LANES + lax.iota(jnp.int32, LANES)
        idx_v[sl] = idx_v[sl] + row * L


def _sc_gather_body(memflat_hbm, vehidx_hbm, out_hbm, idx_v, rows_v, sem):
    wid = lax.axis_index("s") * NC + lax.axis_index("c")
    base = wid * RPW
    _flat_indices(vehidx_hbm, idx_v, base)
    pltpu.async_copy(memflat_hbm.at[idx_v], rows_v, sem).wait()
    pltpu.sync_copy(rows_v, out_hbm.at[pl.ds(base, RPW)])


_sc_gather = functools.partial(
    pl.kernel,
    mesh=_MESH,
    out_type=jax.ShapeDtypeStruct((N, H), jnp.float32),
    scratch_types=[
        pltpu.VMEM((RPW,), jnp.int32),
        pltpu.VMEM((RPW, H), jnp.float32),
        pltpu.SemaphoreType.DMA,
    ],
)(_sc_gather_body)


def _sc_copy_scatter_body(memflat_hbm, nexth_hbm, vehidx_hbm, out_hbm,
                          idx_v, rows_v, copy_sem, sem):
    wid = lax.axis_index("s") * NC + lax.axis_index("c")
    base = wid * RPW
    slab = pl.ds(base * L, RPW * L)
    copy = pltpu.async_copy(memflat_hbm.at[slab], out_hbm.at[slab], copy_sem)
    _flat_indices(vehidx_hbm, idx_v, base)
    pltpu.sync_copy(nexth_hbm.at[pl.ds(base, RPW)], rows_v)
    copy.wait()
    pltpu.async_copy(rows_v, out_hbm.at[idx_v], sem).wait()


_sc_copy_scatter = functools.partial(
    pl.kernel,
    mesh=_MESH,
    out_type=jax.ShapeDtypeStruct((N * L, H), jnp.float32),
    scratch_types=[
        pltpu.VMEM((RPW,), jnp.int32),
        pltpu.VMEM((RPW, H), jnp.float32),
        pltpu.SemaphoreType.DMA,
        pltpu.SemaphoreType.DMA,
    ],
)(_sc_copy_scatter_body)


def _tc_body(curh_ref, x_ref, w_in_t_ref, w_h_t_ref, bias_ref,
             gamma_ref, beta_ref, nh_ref):
    pre = (jnp.dot(x_ref[...], w_in_t_ref[...], preferred_element_type=jnp.float32)
           + jnp.dot(curh_ref[...], w_h_t_ref[...], preferred_element_type=jnp.float32)
           + bias_ref[...])
    mean = jnp.mean(pre, axis=-1, keepdims=True)
    cent = pre - mean
    var = jnp.mean(cent * cent, axis=-1, keepdims=True)
    nh_ref[...] = jnp.tanh(cent * lax.rsqrt(var + 1e-5) * gamma_ref[...]
                           + beta_ref[...])


def kernel(memory, veh_idx, veh_repr, cust_repr, edge_emb,
           W_in, b_in, W_h, b_h, ln_gamma, ln_beta):
    n, l, h = memory.shape
    d = veh_repr.shape[-1]
    x = jnp.concatenate(
        [veh_repr[:, 0, :], cust_repr[:, 0, :], edge_emb[:, 0, 0, :]], axis=-1)
    w_in_t = W_in.T
    w_h_t = W_h.T
    bias = (b_in + b_h).reshape(1, h)
    gamma = ln_gamma.reshape(1, h)
    beta = ln_beta.reshape(1, h)
    vehflat = veh_idx.reshape(n).astype(jnp.int32)
    memflat = memory.reshape(n * l, h)

    cur_h = _sc_gather(memflat, vehflat)

    next_h = pl.pallas_call(
        _tc_body,
        grid=(n // BLOCK_N,),
        in_specs=[
            pl.BlockSpec((BLOCK_N, h), lambda i: (i, 0)),
            pl.BlockSpec((BLOCK_N, 3 * d), lambda i: (i, 0)),
            pl.BlockSpec((3 * d, h), lambda i: (0, 0)),
            pl.BlockSpec((h, h), lambda i: (0, 0)),
            pl.BlockSpec((1, h), lambda i: (0, 0)),
            pl.BlockSpec((1, h), lambda i: (0, 0)),
            pl.BlockSpec((1, h), lambda i: (0, 0)),
        ],
        out_specs=pl.BlockSpec((BLOCK_N, h), lambda i: (i, 0)),
        out_shape=jax.ShapeDtypeStruct((n, h), jnp.float32),
    )(cur_h, x, w_in_t, w_h_t, bias, gamma, beta)

    outflat = _sc_copy_scatter(memflat, next_h, vehflat)
    return outflat.reshape(n, l, h)


# speedup vs baseline: 8.6336x; 8.6336x over previous
import functools

import jax
import jax.numpy as jnp
from jax import lax
from jax.experimental import pallas as pl
from jax.experimental.pallas import tpu as pltpu
from jax.experimental.pallas import tpu_sc as plsc

N, L, H = 4096, 50, 128
NC, NS, LANES = 2, 16, 16
NW = NC * NS
RPW = N // NW

BLOCK_N = 512

_MESH = plsc.VectorSubcoreMesh(core_axis_name="c", subcore_axis_name="s")


def _flat_indices(vehidx_hbm, idx_v, base):
    pltpu.sync_copy(vehidx_hbm.at[pl.ds(base, RPW)], idx_v)
    for g in range(RPW // LANES):
        sl = pl.ds(g * LANES, LANES)
        row = base + g * LANES + lax.iota(jnp.int32, LANES)
        idx_v[sl] = idx_v[sl] + row * L


def _sc_gather_body(memflat_hbm, vehidx_hbm, out_hbm, idx_v, rows_v, sem):
    wid = lax.axis_index("s") * NC + lax.axis_index("c")
    base = wid * RPW
    _flat_indices(vehidx_hbm, idx_v, base)
    pltpu.async_copy(memflat_hbm.at[idx_v], rows_v, sem).wait()
    pltpu.sync_copy(rows_v, out_hbm.at[pl.ds(base, RPW)])


_sc_gather = functools.partial(
    pl.kernel,
    mesh=_MESH,
    out_type=jax.ShapeDtypeStruct((N, H), jnp.float32),
    scratch_types=[
        pltpu.VMEM((RPW,), jnp.int32),
        pltpu.VMEM((RPW, H), jnp.float32),
        pltpu.SemaphoreType.DMA,
    ],
)(_sc_gather_body)


def _sc_scatter_body(outflat_ref, nexth_hbm, vehidx_hbm, idx_v, rows_v, sem):
    wid = lax.axis_index("s") * NC + lax.axis_index("c")
    base = wid * RPW
    _flat_indices(vehidx_hbm, idx_v, base)
    pltpu.sync_copy(nexth_hbm.at[pl.ds(base, RPW)], rows_v)
    pltpu.async_copy(rows_v, outflat_ref.at[idx_v], sem).wait()


_sc_scatter = functools.partial(
    pl.kernel,
    mesh=_MESH,
    out_type=(),
    scratch_types=[
        pltpu.VMEM((RPW,), jnp.int32),
        pltpu.VMEM((RPW, H), jnp.float32),
        pltpu.SemaphoreType.DMA,
    ],
)(_sc_scatter_body)


def _tc_body(curh_ref, x_ref, w_in_t_ref, w_h_t_ref, bias_ref,
             gamma_ref, beta_ref, nh_ref):
    pre = (jnp.dot(x_ref[...], w_in_t_ref[...], preferred_element_type=jnp.float32)
           + jnp.dot(curh_ref[...], w_h_t_ref[...], preferred_element_type=jnp.float32)
           + bias_ref[...])
    mean = jnp.mean(pre, axis=-1, keepdims=True)
    cent = pre - mean
    var = jnp.mean(cent * cent, axis=-1, keepdims=True)
    nh_ref[...] = jnp.tanh(cent * lax.rsqrt(var + 1e-5) * gamma_ref[...]
                           + beta_ref[...])


def kernel(memory, veh_idx, veh_repr, cust_repr, edge_emb,
           W_in, b_in, W_h, b_h, ln_gamma, ln_beta):
    n, l, h = memory.shape
    d = veh_repr.shape[-1]
    x = jnp.concatenate(
        [veh_repr[:, 0, :], cust_repr[:, 0, :], edge_emb[:, 0, 0, :]], axis=-1)
    w_in_t = W_in.T
    w_h_t = W_h.T
    bias = (b_in + b_h).reshape(1, h)
    gamma = ln_gamma.reshape(1, h)
    beta = ln_beta.reshape(1, h)
    vehflat = veh_idx.reshape(n).astype(jnp.int32)
    mf_ref = jax.new_ref(memory.reshape(n * l, h))

    cur_h = _sc_gather(mf_ref, vehflat)

    next_h = pl.pallas_call(
        _tc_body,
        grid=(n // BLOCK_N,),
        in_specs=[
            pl.BlockSpec((BLOCK_N, h), lambda i: (i, 0)),
            pl.BlockSpec((BLOCK_N, 3 * d), lambda i: (i, 0)),
            pl.BlockSpec((3 * d, h), lambda i: (0, 0)),
            pl.BlockSpec((h, h), lambda i: (0, 0)),
            pl.BlockSpec((1, h), lambda i: (0, 0)),
            pl.BlockSpec((1, h), lambda i: (0, 0)),
            pl.BlockSpec((1, h), lambda i: (0, 0)),
        ],
        out_specs=pl.BlockSpec((BLOCK_N, h), lambda i: (i, 0)),
        out_shape=jax.ShapeDtypeStruct((n, h), jnp.float32),
    )(cur_h, x, w_in_t, w_h_t, bias, gamma, beta)

    _sc_scatter(mf_ref, next_h, vehflat)
    return jax.freeze(mf_ref).reshape(n, l, h)


# speedup vs baseline: 15.8490x; 1.8357x over previous
import jax
import jax.numpy as jnp
from jax import lax
from jax.experimental import pallas as pl
from jax.experimental.pallas import tpu as pltpu

BLOCK_N = 256


def _fused_body(idx_sref, mem_ref, x_ref, w_in_t_ref, w_h_t_ref, bias_ref,
                gamma_ref, beta_ref, out_ref, curh_scr, nh_scr):
    i = pl.program_id(0)
    b = mem_ref.shape[0]
    out_ref[...] = mem_ref[...]

    def gather_row(r, _):
        idx = idx_sref[i * b + r]
        curh_scr[r, :] = mem_ref[r, idx, :]
        return 0

    lax.fori_loop(0, b, gather_row, 0, unroll=8)

    pre = (jnp.dot(x_ref[...], w_in_t_ref[...], preferred_element_type=jnp.float32)
           + jnp.dot(curh_scr[...], w_h_t_ref[...], preferred_element_type=jnp.float32)
           + bias_ref[...])
    mean = jnp.mean(pre, axis=-1, keepdims=True)
    cent = pre - mean
    var = jnp.mean(cent * cent, axis=-1, keepdims=True)
    nh_scr[...] = jnp.tanh(cent * lax.rsqrt(var + 1e-5) * gamma_ref[...]
                           + beta_ref[...])

    def scatter_row(r, _):
        idx = idx_sref[i * b + r]
        out_ref[r, idx, :] = nh_scr[r, :]
        return 0

    lax.fori_loop(0, b, scatter_row, 0, unroll=8)


def kernel(memory, veh_idx, veh_repr, cust_repr, edge_emb,
           W_in, b_in, W_h, b_h, ln_gamma, ln_beta):
    n, l, h = memory.shape
    d = veh_repr.shape[-1]
    x = jnp.concatenate(
        [veh_repr[:, 0, :], cust_repr[:, 0, :], edge_emb[:, 0, 0, :]], axis=-1)
    w_in_t = W_in.T
    w_h_t = W_h.T
    bias = (b_in + b_h).reshape(1, h)
    gamma = ln_gamma.reshape(1, h)
    beta = ln_beta.reshape(1, h)
    idx = veh_idx.reshape(n).astype(jnp.int32)

    grid_spec = pltpu.PrefetchScalarGridSpec(
        num_scalar_prefetch=1,
        grid=(n // BLOCK_N,),
        in_specs=[
            pl.BlockSpec((BLOCK_N, l, h), lambda i, *_: (i, 0, 0)),
            pl.BlockSpec((BLOCK_N, 3 * d), lambda i, *_: (i, 0)),
            pl.BlockSpec((3 * d, h), lambda i, *_: (0, 0)),
            pl.BlockSpec((h, h), lambda i, *_: (0, 0)),
            pl.BlockSpec((1, h), lambda i, *_: (0, 0)),
            pl.BlockSpec((1, h), lambda i, *_: (0, 0)),
            pl.BlockSpec((1, h), lambda i, *_: (0, 0)),
        ],
        out_specs=pl.BlockSpec((BLOCK_N, l, h), lambda i, *_: (i, 0, 0)),
        scratch_shapes=[
            pltpu.VMEM((BLOCK_N, h), jnp.float32),
            pltpu.VMEM((BLOCK_N, h), jnp.float32),
        ],
    )
    return pl.pallas_call(
        _fused_body,
        grid_spec=grid_spec,
        out_shape=jax.ShapeDtypeStruct((n, l, h), jnp.float32),
    )(idx, memory, x, w_in_t, w_h_t, bias, gamma, beta)
